# baseline (speedup 1.0000x reference)
import functools
import math

import jax
import jax.numpy as jnp
from jax import lax
from jax.experimental import pallas as pl
from jax.experimental.pallas import tpu as pltpu
from jax.experimental.pallas import tpu_sc as plsc

N, D, H, NE, K = 2048, 768, 768, 64, 8
TEMP_INV = 1.0 / math.e
NEG_INF = -1e30

B = 256
NBLK = 128
SPAD = NBLK * B
NW = 32
TPT = N // NW
CHUNK = 8
WREP = 128


def _router_body(x_ref, gw_ref, gb_ref, slot_ref, w_ref, counts_ref):
    scores = lax.dot_general(
        x_ref[...], gw_ref[...], (((1,), (1,)), ((), ())),
        preferred_element_type=jnp.float32)
    scores = scores * TEMP_INV + gb_ref[...]
    m = jnp.max(scores, axis=1, keepdims=True)
    p = jnp.exp(scores - m)
    p = p / jnp.sum(p, axis=1, keepdims=True)

    e_iota = lax.broadcasted_iota(jnp.int32, (N, NE), 1)
    work = p
    sel_total = jnp.zeros((N, NE), jnp.float32)
    idx_cols = []
    val_cols = []
    for _ in range(K):
        mk = jnp.max(work, axis=1, keepdims=True)
        cand = jnp.where(work == mk, e_iota, NE)
        idx = jnp.min(cand, axis=1, keepdims=True)
        sel = (e_iota == idx)
        sel_total = sel_total + sel.astype(jnp.float32)
        idx_cols.append(idx)
        val_cols.append(mk)
        work = jnp.where(sel, NEG_INF, work)

    denom = jnp.sum(jnp.concatenate(val_cols, axis=1), axis=1,
                    keepdims=True) + 1e-8

    counts_f = jnp.sum(sel_total, axis=0, keepdims=True)
    counts_i = (counts_f + 0.5).astype(jnp.int32)
    nblk = (counts_i + (B - 1)) // B
    cpad_f = (nblk * B).astype(jnp.float32)
    f_lt_e = (lax.broadcasted_iota(jnp.int32, (NE, NE), 0) <
              lax.broadcasted_iota(jnp.int32, (NE, NE), 1))
    off_f = lax.dot_general(
        cpad_f, f_lt_e.astype(jnp.float32), (((1,), (0,)), ((), ())),
        preferred_element_type=jnp.float32)

    excl_chunks = []
    rows = N // CHUNK
    for c in range(CHUNK):
        row_i = lax.broadcasted_iota(jnp.int32, (rows, N), 0) + c * rows
        col_i = lax.broadcasted_iota(jnp.int32, (rows, N), 1)
        tri = (col_i < row_i).astype(jnp.float32)
        excl_chunks.append(lax.dot_general(
            tri, sel_total, (((1,), (0,)), ((), ())),
            preferred_element_type=jnp.float32))
    excl = jnp.concatenate(excl_chunks, axis=0)

    slot_all = excl + off_f
    slot_cols = []
    w_cols = []
    for k in range(K):
        sel = (e_iota == idx_cols[k])
        slot_k = jnp.sum(jnp.where(sel, slot_all, 0.0), axis=1,
                         keepdims=True)
        slot_cols.append((slot_k + 0.5).astype(jnp.int32))
        w_cols.append(val_cols[k] / denom)

    slot_ref[...] = jnp.concatenate(slot_cols, axis=1)
    w_ref[...] = jnp.concatenate(w_cols, axis=1)
    counts_ref[...] = counts_i


def _router(x, gate_w, gate_b):
    return pl.pallas_call(
        _router_body,
        out_shape=(
            jax.ShapeDtypeStruct((N, K), jnp.int32),
            jax.ShapeDtypeStruct((N, K), jnp.float32),
            jax.ShapeDtypeStruct((1, NE), jnp.int32),
        ),
    )(x, gate_w, gate_b.reshape(1, NE))


def _dispatch_body(x_hbm, slotR_hbm, xg_hbm, *scratch):
    idx_bufs = scratch[:K]
    xbuf, sem = scratch[K], scratch[K + 1]
    wid = lax.axis_index("s") * 2 + lax.axis_index("c")
    base = wid * TPT
    pltpu.sync_copy(x_hbm.at[pl.ds(base, TPT)], xbuf)
    for k in range(K):
        pltpu.sync_copy(slotR_hbm.at[wid, k], idx_bufs[k])
    copies = []
    for k in range(K):
        copies.append(pltpu.async_copy(xbuf, xg_hbm.at[idx_bufs[k]], sem))
    for c in copies:
        c.wait()


def _dispatch(x, slotR):
    mesh = plsc.VectorSubcoreMesh(core_axis_name="c", subcore_axis_name="s")
    return pl.kernel(
        _dispatch_body,
        out_type=jax.ShapeDtypeStruct((SPAD, D), jnp.float32),
        mesh=mesh,
        scratch_types=(
            [pltpu.VMEM((TPT,), jnp.int32) for _ in range(K)] + [
                pltpu.VMEM((TPT, D), jnp.float32),
                pltpu.SemaphoreType.DMA,
            ]),
    )(x, slotR)


def _gmm_body(be_ref, nt_ref, xg_ref, W1_ref, b1_ref, W2_ref,
              b2_ref, ys_ref):
    t = pl.program_id(0)

    @pl.when(t < nt_ref[0])
    def _():
        h = lax.dot_general(
            xg_ref[...], W1_ref[0], (((1,), (1,)), ((), ())),
            preferred_element_type=jnp.float32)
        h = jnp.maximum(h + b1_ref[0, 0, :], 0.0)
        y = lax.dot_general(
            h, W2_ref[0], (((1,), (1,)), ((), ())),
            preferred_element_type=jnp.float32)
        ys_ref[...] = y + b2_ref[0, 0, :]


def _tm(t, nt):
    return jnp.minimum(t, nt[0] - 1)


def _gmm(be, nt, xg, W1, b1, W2, b2):
    grid_spec = pltpu.PrefetchScalarGridSpec(
        num_scalar_prefetch=2,
        grid=(NBLK,),
        in_specs=[
            pl.BlockSpec((B, D), lambda t, be, nt: (_tm(t, nt), 0)),
            pl.BlockSpec((1, H, D),
                         lambda t, be, nt: (be[_tm(t, nt)], 0, 0)),
            pl.BlockSpec((1, 1, H),
                         lambda t, be, nt: (be[_tm(t, nt)], 0, 0)),
            pl.BlockSpec((1, D, H),
                         lambda t, be, nt: (be[_tm(t, nt)], 0, 0)),
            pl.BlockSpec((1, 1, D),
                         lambda t, be, nt: (be[_tm(t, nt)], 0, 0)),
        ],
        out_specs=pl.BlockSpec((B, D), lambda t, be, nt: (_tm(t, nt), 0)),
    )
    return pl.pallas_call(
        _gmm_body,
        grid_spec=grid_spec,
        out_shape=jax.ShapeDtypeStruct((SPAD, D), jnp.float32),
    )(be, nt, xg, W1, b1.reshape(NE, 1, H), W2, b2.reshape(NE, 1, D))


def _combine_body(ys_hbm, slotR_hbm, wR_hbm, out_hbm, *scratch):
    idx_bufs = scratch[:K]
    w_bufs = scratch[K:2 * K]
    gbufs = scratch[2 * K:2 * K + 2]
    acc = scratch[2 * K + 2]
    sems = scratch[2 * K + 3:2 * K + 5]
    wid = lax.axis_index("s") * 2 + lax.axis_index("c")
    base = wid * TPT
    HT = TPT // 2
    for k in range(K):
        pltpu.sync_copy(slotR_hbm.at[wid, k], idx_bufs[k])
        pltpu.sync_copy(wR_hbm.at[wid, k], w_bufs[k])

    def fire(j):
        half, k = divmod(j, K)
        return pltpu.async_copy(
            ys_hbm.at[idx_bufs[k].at[pl.ds(half * HT, HT)]],
            gbufs[j % 2], sems[j % 2])

    cp = fire(0)
    for j in range(2 * K):
        half, k = divmod(j, K)
        cp.wait()
        if j + 1 < 2 * K:
            cp = fire(j + 1)
        g2 = gbufs[j % 2]

        def row_body(r, _, k=k, half=half, g2=g2):
            w = w_bufs[k][pl.ds(r, 16)][0]
            a = acc.at[half * HT + r]
            g = g2.at[r]
            for c in range(D // 16):
                s = pl.ds(c * 16, 16)
                if k == 0:
                    a[s] = g[s] * w
                else:
                    a[s] = a[s] + g[s] * w
            return 0

        lax.fori_loop(0, HT, row_body, 0)
    pltpu.sync_copy(acc, out_hbm.at[pl.ds(base, TPT)])


def _combine(ys, slotR, wR):
    mesh = plsc.VectorSubcoreMesh(core_axis_name="c", subcore_axis_name="s")
    return pl.kernel(
        _combine_body,
        out_type=jax.ShapeDtypeStruct((N, D), jnp.float32),
        mesh=mesh,
        scratch_types=(
            [pltpu.VMEM((TPT,), jnp.int32) for _ in range(K)] +
            [pltpu.VMEM((TPT + 16,), jnp.float32) for _ in range(K)] + [
                pltpu.VMEM((TPT // 2, D), jnp.float32),
                pltpu.VMEM((TPT // 2, D), jnp.float32),
                pltpu.VMEM((TPT, D), jnp.float32),
                pltpu.SemaphoreType.DMA,
                pltpu.SemaphoreType.DMA,
            ]),
    )(ys, slotR, wR)


@jax.jit
def kernel(x, gate_w, gate_b, W1, b1, W2, b2):
    slot_nk, w_nk, counts = _router(x, gate_w, gate_b)
    slotR = slot_nk.reshape(NW, TPT, K).transpose(0, 2, 1)
    wR = jnp.pad(w_nk.reshape(NW, TPT, K).transpose(0, 2, 1),
                 ((0, 0), (0, 0), (0, 16)))
    nblk = (counts[0] + (B - 1)) // B
    be = jnp.minimum(
        jnp.repeat(jnp.arange(NE, dtype=jnp.int32), nblk,
                   total_repeat_length=NBLK), NE - 1)
    nt = jnp.sum(nblk).reshape(1)
    xg = _dispatch(x, slotR)
    ys = _gmm(be, nt, xg, W1, b1, W2, b2)
    return _combine(ys, slotR, wR)


# speedup vs baseline: 1.1500x; 1.1500x over previous
import functools
import math

import jax
import jax.numpy as jnp
from jax import lax
from jax.experimental import pallas as pl
from jax.experimental.pallas import tpu as pltpu
from jax.experimental.pallas import tpu_sc as plsc

N, D, H, NE, K = 2048, 768, 768, 64, 8
TEMP_INV = 1.0 / math.e
NEG_INF = -1e30

B = 256
NBLK = 128
SPAD = NBLK * B
NW = 32
TPT = N // NW
CHUNK = 8
WREP = 128


def _router_body(x_ref, gw_ref, gb_ref, slot_ref, w_ref, counts_ref):
    scores = lax.dot_general(
        x_ref[...], gw_ref[...], (((1,), (1,)), ((), ())),
        preferred_element_type=jnp.float32)
    scores = scores * TEMP_INV + gb_ref[...]
    m = jnp.max(scores, axis=1, keepdims=True)
    p = jnp.exp(scores - m)
    p = p / jnp.sum(p, axis=1, keepdims=True)

    e_iota = lax.broadcasted_iota(jnp.int32, (N, NE), 1)
    work = p
    sel_total = jnp.zeros((N, NE), jnp.float32)
    idx_cols = []
    val_cols = []
    for _ in range(K):
        mk = jnp.max(work, axis=1, keepdims=True)
        cand = jnp.where(work == mk, e_iota, NE)
        idx = jnp.min(cand, axis=1, keepdims=True)
        sel = (e_iota == idx)
        sel_total = sel_total + sel.astype(jnp.float32)
        idx_cols.append(idx)
        val_cols.append(mk)
        work = jnp.where(sel, NEG_INF, work)

    denom = jnp.sum(jnp.concatenate(val_cols, axis=1), axis=1,
                    keepdims=True) + 1e-8

    counts_f = jnp.sum(sel_total, axis=0, keepdims=True)
    counts_i = (counts_f + 0.5).astype(jnp.int32)
    nblk = (counts_i + (B - 1)) // B
    cpad_f = (nblk * B).astype(jnp.float32)
    f_lt_e = (lax.broadcasted_iota(jnp.int32, (NE, NE), 0) <
              lax.broadcasted_iota(jnp.int32, (NE, NE), 1))
    off_f = lax.dot_general(
        cpad_f, f_lt_e.astype(jnp.float32), (((1,), (0,)), ((), ())),
        preferred_element_type=jnp.float32)

    excl_chunks = []
    rows = N // CHUNK
    for c in range(CHUNK):
        row_i = lax.broadcasted_iota(jnp.int32, (rows, N), 0) + c * rows
        col_i = lax.broadcasted_iota(jnp.int32, (rows, N), 1)
        tri = (col_i < row_i).astype(jnp.float32)
        excl_chunks.append(lax.dot_general(
            tri, sel_total, (((1,), (0,)), ((), ())),
            preferred_element_type=jnp.float32))
    excl = jnp.concatenate(excl_chunks, axis=0)

    slot_all = excl + off_f
    slot_cols = []
    w_cols = []
    for k in range(K):
        sel = (e_iota == idx_cols[k])
        slot_k = jnp.sum(jnp.where(sel, slot_all, 0.0), axis=1,
                         keepdims=True)
        slot_cols.append((slot_k + 0.5).astype(jnp.int32))
        w_cols.append(val_cols[k] / denom)

    slot_ref[...] = jnp.concatenate(slot_cols, axis=1)
    w_ref[...] = jnp.concatenate(w_cols, axis=1)
    counts_ref[...] = counts_i


def _router(x, gate_w, gate_b):
    return pl.pallas_call(
        _router_body,
        out_shape=(
            jax.ShapeDtypeStruct((N, K), jnp.int32),
            jax.ShapeDtypeStruct((N, K), jnp.float32),
            jax.ShapeDtypeStruct((1, NE), jnp.int32),
        ),
    )(x, gate_w, gate_b.reshape(1, NE))


def _dispatch_body(x_hbm, slotR_hbm, xg_hbm, *scratch):
    idx_bufs = scratch[:K]
    xbuf, sem = scratch[K], scratch[K + 1]
    wid = lax.axis_index("s") * 2 + lax.axis_index("c")
    base = wid * TPT
    pltpu.sync_copy(x_hbm.at[pl.ds(base, TPT)], xbuf)
    for k in range(K):
        pltpu.sync_copy(slotR_hbm.at[wid, k], idx_bufs[k])
    copies = []
    for k in range(K):
        copies.append(pltpu.async_copy(xbuf, xg_hbm.at[idx_bufs[k]], sem))
    for c in copies:
        c.wait()


def _dispatch(x, slotR):
    mesh = plsc.VectorSubcoreMesh(core_axis_name="c", subcore_axis_name="s")
    return pl.kernel(
        _dispatch_body,
        out_type=jax.ShapeDtypeStruct((SPAD, D), jnp.float32),
        mesh=mesh,
        scratch_types=(
            [pltpu.VMEM((TPT,), jnp.int32) for _ in range(K)] + [
                pltpu.VMEM((TPT, D), jnp.float32),
                pltpu.SemaphoreType.DMA,
            ]),
    )(x, slotR)


def _gmm_body(be_ref, nt_ref, xg_ref, W1_ref, b1_ref, W2_ref,
              b2_ref, ys_ref):
    t = pl.program_id(0)

    @pl.when(t < nt_ref[0])
    def _():
        h = lax.dot_general(
            xg_ref[...], W1_ref[0], (((1,), (1,)), ((), ())),
            preferred_element_type=jnp.float32)
        h = jnp.maximum(h + b1_ref[0, 0, :], 0.0)
        y = lax.dot_general(
            h, W2_ref[0], (((1,), (1,)), ((), ())),
            preferred_element_type=jnp.float32)
        ys_ref[...] = y + b2_ref[0, 0, :]


def _tm(t, nt):
    return jnp.minimum(t, nt[0] - 1)


def _gmm(be, nt, xg, W1, b1, W2, b2):
    grid_spec = pltpu.PrefetchScalarGridSpec(
        num_scalar_prefetch=2,
        grid=(NBLK,),
        in_specs=[
            pl.BlockSpec((B, D), lambda t, be, nt: (_tm(t, nt), 0)),
            pl.BlockSpec((1, H, D),
                         lambda t, be, nt: (be[_tm(t, nt)], 0, 0)),
            pl.BlockSpec((1, 1, H),
                         lambda t, be, nt: (be[_tm(t, nt)], 0, 0)),
            pl.BlockSpec((1, D, H),
                         lambda t, be, nt: (be[_tm(t, nt)], 0, 0)),
            pl.BlockSpec((1, 1, D),
                         lambda t, be, nt: (be[_tm(t, nt)], 0, 0)),
        ],
        out_specs=pl.BlockSpec((B, D), lambda t, be, nt: (_tm(t, nt), 0)),
    )
    return pl.pallas_call(
        _gmm_body,
        grid_spec=grid_spec,
        out_shape=jax.ShapeDtypeStruct((SPAD, D), jnp.float32),
    )(be, nt, xg, W1, b1.reshape(NE, 1, H), W2, b2.reshape(NE, 1, D))


def _combine_body(ys_hbm, slotR_hbm, wR_hbm, out_hbm, *scratch):
    idx_bufs = scratch[:2 * K]
    w_bufs = scratch[2 * K:3 * K]
    gbufs = scratch[3 * K:3 * K + 4]
    acc = scratch[3 * K + 4]
    sems = scratch[3 * K + 5:3 * K + 9]
    wid = lax.axis_index("s") * 2 + lax.axis_index("c")
    base = wid * TPT
    HT = TPT // 2
    for k in range(K):
        pltpu.sync_copy(wR_hbm.at[wid, k], w_bufs[k])
        for half in range(2):
            pltpu.sync_copy(slotR_hbm.at[wid, k, pl.ds(half * HT, HT)],
                            idx_bufs[half * K + k])

    NP = K // 2
    for half in range(2):
        def fire(jp, half=half):
            bs = (jp % 2) * 2
            c0 = pltpu.async_copy(
                ys_hbm.at[idx_bufs[half * K + 2 * jp]], gbufs[bs],
                sems[bs])
            c1 = pltpu.async_copy(
                ys_hbm.at[idx_bufs[half * K + 2 * jp + 1]], gbufs[bs + 1],
                sems[bs + 1])
            return (c0, c1)

        pend = {0: fire(0), 1: fire(1)}
        for jp in range(NP):
            k0, k1 = 2 * jp, 2 * jp + 1
            pend[jp][0].wait()
            pend[jp][1].wait()
            bs = (jp % 2) * 2
            g0, g1 = gbufs[bs], gbufs[bs + 1]

            def row_body(r, _, k0=k0, k1=k1, half=half, g0=g0, g1=g1,
                         first=(jp == 0)):
                w0 = w_bufs[k0][pl.ds(half * HT + r, 16)][0]
                w1 = w_bufs[k1][pl.ds(half * HT + r, 16)][0]
                a = acc.at[r]
                ga = g0.at[r]
                gb = g1.at[r]
                for c in range(D // 16):
                    s = pl.ds(c * 16, 16)
                    v = ga[s] * w0 + gb[s] * w1
                    if first:
                        a[s] = v
                    else:
                        a[s] = a[s] + v
                return 0

            lax.fori_loop(0, HT, row_body, 0)
            if jp + 2 < NP:
                pend[jp + 2] = fire(jp + 2)
        pltpu.sync_copy(acc, out_hbm.at[pl.ds(base + half * HT, HT)])


def _combine(ys, slotR, wR):
    mesh = plsc.VectorSubcoreMesh(core_axis_name="c", subcore_axis_name="s")
    return pl.kernel(
        _combine_body,
        out_type=jax.ShapeDtypeStruct((N, D), jnp.float32),
        mesh=mesh,
        scratch_types=(
            [pltpu.VMEM((TPT // 2,), jnp.int32) for _ in range(2 * K)] +
            [pltpu.VMEM((TPT + 16,), jnp.float32) for _ in range(K)] +
            [pltpu.VMEM((TPT // 2, D), jnp.float32) for _ in range(4)] + [
                pltpu.VMEM((TPT // 2, D), jnp.float32),
                pltpu.SemaphoreType.DMA,
                pltpu.SemaphoreType.DMA,
                pltpu.SemaphoreType.DMA,
                pltpu.SemaphoreType.DMA,
            ]),
    )(ys, slotR, wR)


@jax.jit
def kernel(x, gate_w, gate_b, W1, b1, W2, b2):
    slot_nk, w_nk, counts = _router(x, gate_w, gate_b)
    slotR = slot_nk.reshape(NW, TPT, K).transpose(0, 2, 1)
    wR = jnp.pad(w_nk.reshape(NW, TPT, K).transpose(0, 2, 1),
                 ((0, 0), (0, 0), (0, 16)))
    nblk = (counts[0] + (B - 1)) // B
    be = jnp.minimum(
        jnp.repeat(jnp.arange(NE, dtype=jnp.int32), nblk,
                   total_repeat_length=NBLK), NE - 1)
    nt = jnp.sum(nblk).reshape(1)
    xg = _dispatch(x, slotR)
    ys = _gmm(be, nt, xg, W1, b1, W2, b2)
    return _combine(ys, slotR, wR)
